# baseline (speedup 1.0000x reference)
import jax
import jax.numpy as jnp
from jax.experimental import pallas as pl
from jax.experimental.pallas import tpu as pltpu

Q = 5


def _pick_tile(n, cap=400):
    best = 8
    for t in range(8, cap + 1, 8):
        if n % t == 0:
            best = t
    return best


def _sums_kernel(*refs):
    t_refs = refs[:Q]
    row_ref, col_ref = refs[Q], refs[Q + 1]
    ti = t_refs[0].shape[0]
    col = None
    for q in range(Q):
        blk = t_refs[q][...]
        row_ref[pl.ds(q * ti, ti), :] = jnp.sum(blk, axis=1, keepdims=True)
        c = jnp.sum(blk, axis=0)
        col = c if col is None else col + c
    col_ref[...] = col[None, None, :]


def _hs_kernel(x_ref, w1t_ref, b1_ref, d1_ref, d2_ref, hs_ref, s_ref):
    s = jnp.sqrt(d1_ref[...] * d2_ref[...])
    h = jnp.dot(x_ref[...], w1t_ref[...], preferred_element_type=jnp.float32)
    hs_ref[...] = s * (h + b1_ref[...])
    s_ref[...] = s


def _spmm1_kernel(*refs):
    t_refs = refs[:Q]
    hs_ref, w2t_ref, b2_ref, s_ref, z_ref = refs[Q:]
    ti = t_refs[0].shape[0]
    hs = hs_ref[...]
    w2t = w2t_ref[...]
    b2 = b2_ref[...]
    for q in range(Q):
        t = jnp.dot(t_refs[q][...], hs, preferred_element_type=jnp.float32)
        r = jnp.maximum(t, 0.0)
        z = jnp.dot(r, w2t, preferred_element_type=jnp.float32) + b2
        z_ref[pl.ds(q * ti, ti), :] = z * s_ref[pl.ds(q * ti, ti), :]


def _spmm2_kernel(*refs):
    t_refs = refs[:Q]
    z_ref, o_ref = refs[Q], refs[Q + 1]
    ti = t_refs[0].shape[0]
    z = z_ref[...]
    for q in range(Q):
        o_ref[pl.ds(q * ti, ti), :] = jnp.dot(
            t_refs[q][...], z, preferred_element_type=jnp.float32)


def _tilde_specs(ti, n):
    return [
        pl.BlockSpec((ti, n), lambda i, q=q: (i * Q + q, 0))
        for q in range(Q)
    ]


def kernel(X, tilde, W1, b1, W2, b2):
    n, feat = X.shape
    hid = W1.shape[0]
    ncls = W2.shape[0]
    tt = _pick_tile(n)
    ti = tt // Q
    nb = n // tt

    row, colpart = pl.pallas_call(
        _sums_kernel,
        grid=(nb,),
        in_specs=_tilde_specs(ti, n),
        out_specs=[
            pl.BlockSpec((tt, 1), lambda i: (i, 0)),
            pl.BlockSpec((1, 1, n), lambda i: (i, 0, 0)),
        ],
        out_shape=[
            jax.ShapeDtypeStruct((n, 1), jnp.float32),
            jax.ShapeDtypeStruct((nb, 1, n), jnp.float32),
        ],
        compiler_params=pltpu.CompilerParams(
            dimension_semantics=("parallel",),
        ),
    )(*([tilde] * Q))

    d1 = jnp.sum(colpart, axis=(0, 1)).reshape(n, 1)

    hs, s = pl.pallas_call(
        _hs_kernel,
        out_shape=[
            jax.ShapeDtypeStruct((n, hid), jnp.float32),
            jax.ShapeDtypeStruct((n, 1), jnp.float32),
        ],
    )(X, W1.T, b1.reshape(1, hid), d1, row)

    z = pl.pallas_call(
        _spmm1_kernel,
        grid=(nb,),
        in_specs=_tilde_specs(ti, n) + [
            pl.BlockSpec((n, hid), lambda i: (0, 0)),
            pl.BlockSpec((hid, ncls), lambda i: (0, 0)),
            pl.BlockSpec((1, ncls), lambda i: (0, 0)),
            pl.BlockSpec((tt, 1), lambda i: (i, 0)),
        ],
        out_specs=pl.BlockSpec((tt, ncls), lambda i: (i, 0)),
        out_shape=jax.ShapeDtypeStruct((n, ncls), jnp.float32),
        compiler_params=pltpu.CompilerParams(
            dimension_semantics=("parallel",),
        ),
    )(*([tilde] * Q), hs, W2.T, b2.reshape(1, ncls), s)

    o = pl.pallas_call(
        _spmm2_kernel,
        grid=(nb,),
        in_specs=_tilde_specs(ti, n) + [
            pl.BlockSpec((n, ncls), lambda i: (0, 0)),
        ],
        out_specs=pl.BlockSpec((tt, ncls), lambda i: (i, 0)),
        out_shape=jax.ShapeDtypeStruct((n, ncls), jnp.float32),
        compiler_params=pltpu.CompilerParams(
            dimension_semantics=("parallel",),
        ),
    )(*([tilde] * Q), z)
    return o


# speedup vs baseline: 1.0945x; 1.0945x over previous
import jax
import jax.numpy as jnp
from jax.experimental import pallas as pl
from jax.experimental.pallas import tpu as pltpu


def _pick_tile(n, cap=400):
    best = 16
    for t in range(16, cap + 1, 16):
        if n % t == 0:
            best = t
    return best


def _sums_cast_kernel(t_ref, row_ref, col_ref, tb_ref):
    blk = t_ref[...]
    row_ref[...] = jnp.sum(blk, axis=1, keepdims=True)
    col_ref[...] = jnp.sum(blk, axis=0)[None, None, :]
    tb_ref[...] = blk.astype(jnp.bfloat16)


def _hs_kernel(x_ref, w1t_ref, b1_ref, d1_ref, d2_ref, hs_ref, s_ref):
    s = jnp.sqrt(d1_ref[...] * d2_ref[...])
    h = jnp.dot(x_ref[...], w1t_ref[...], preferred_element_type=jnp.float32)
    hs_ref[...] = (s * (h + b1_ref[...])).astype(jnp.bfloat16)
    s_ref[...] = s


def _spmm1_kernel(t_ref, hs_ref, w2t_ref, b2_ref, s_ref, z_ref):
    t = jnp.dot(t_ref[...], hs_ref[...], preferred_element_type=jnp.float32)
    r = jnp.maximum(t, 0.0)
    z = jnp.dot(r, w2t_ref[...], preferred_element_type=jnp.float32) + b2_ref[...]
    z_ref[...] = (z * s_ref[...]).astype(jnp.bfloat16)


def _spmm2_kernel(t_ref, z_ref, o_ref):
    o_ref[...] = jnp.dot(t_ref[...], z_ref[...], preferred_element_type=jnp.float32)


def kernel(X, tilde, W1, b1, W2, b2):
    n, feat = X.shape
    hid = W1.shape[0]
    ncls = W2.shape[0]
    ti = _pick_tile(n)
    nb = n // ti

    row, colpart, tb = pl.pallas_call(
        _sums_cast_kernel,
        grid=(nb,),
        in_specs=[pl.BlockSpec((ti, n), lambda i: (i, 0))],
        out_specs=[
            pl.BlockSpec((ti, 1), lambda i: (i, 0)),
            pl.BlockSpec((1, 1, n), lambda i: (i, 0, 0)),
            pl.BlockSpec((ti, n), lambda i: (i, 0)),
        ],
        out_shape=[
            jax.ShapeDtypeStruct((n, 1), jnp.float32),
            jax.ShapeDtypeStruct((nb, 1, n), jnp.float32),
            jax.ShapeDtypeStruct((n, n), jnp.bfloat16),
        ],
        compiler_params=pltpu.CompilerParams(
            dimension_semantics=("parallel",),
        ),
    )(tilde)

    d1 = jnp.sum(colpart, axis=(0, 1)).reshape(n, 1)

    hs, s = pl.pallas_call(
        _hs_kernel,
        out_shape=[
            jax.ShapeDtypeStruct((n, hid), jnp.bfloat16),
            jax.ShapeDtypeStruct((n, 1), jnp.float32),
        ],
    )(X, W1.T, b1.reshape(1, hid), d1, row)

    z = pl.pallas_call(
        _spmm1_kernel,
        grid=(nb,),
        in_specs=[
            pl.BlockSpec((ti, n), lambda i: (i, 0)),
            pl.BlockSpec((n, hid), lambda i: (0, 0)),
            pl.BlockSpec((hid, ncls), lambda i: (0, 0)),
            pl.BlockSpec((1, ncls), lambda i: (0, 0)),
            pl.BlockSpec((ti, 1), lambda i: (i, 0)),
        ],
        out_specs=pl.BlockSpec((ti, ncls), lambda i: (i, 0)),
        out_shape=jax.ShapeDtypeStruct((n, ncls), jnp.bfloat16),
        compiler_params=pltpu.CompilerParams(
            dimension_semantics=("parallel",),
        ),
    )(tb, hs, W2.T, b2.reshape(1, ncls), s)

    o = pl.pallas_call(
        _spmm2_kernel,
        grid=(nb,),
        in_specs=[
            pl.BlockSpec((ti, n), lambda i: (i, 0)),
            pl.BlockSpec((n, ncls), lambda i: (0, 0)),
        ],
        out_specs=pl.BlockSpec((ti, ncls), lambda i: (i, 0)),
        out_shape=jax.ShapeDtypeStruct((n, ncls), jnp.float32),
        compiler_params=pltpu.CompilerParams(
            dimension_semantics=("parallel",),
        ),
    )(tb, z)
    return o
